# baseline (speedup 1.0000x reference)
import functools

import jax
import jax.numpy as jnp
from jax.experimental import pallas as pl
from jax.experimental.pallas import tpu as pltpu

_B = 16
_TB = 8192


def _mipl_body(cu_ref, x_ref, bu_ref, bz_ref,
               out_ref, m_ref, s_ref, n_ref, *, nblocks, tb, nseg):
    step = pl.program_id(0)

    @pl.when(step == 0)
    def _init():
        m_ref[...] = jnp.full_like(m_ref, -1e30)
        s_ref[...] = jnp.zeros_like(s_ref)
        n_ref[...] = jnp.zeros_like(n_ref)

    x = x_ref[...]
    bu = bu_ref[...]
    bz = bz_ref[...]
    eta = bz * jax.lax.rsqrt(jnp.mean(bz * bz, axis=0, keepdims=True))
    xw = jnp.dot(x, bu, preferred_element_type=jnp.float32)
    xt = jnp.dot(x, eta, preferred_element_type=jnp.float32)

    start = step * tb
    gidx = start + jax.lax.broadcasted_iota(jnp.int32, (tb, 1), 0)
    seg = jnp.zeros((tb, 1), jnp.int32)
    for b in range(1, nseg):
        seg += (gidx >= cu_ref[b]).astype(jnp.int32)
    oh = (seg == jax.lax.broadcasted_iota(jnp.int32, (tb, nseg), 1)
          ).astype(jnp.float32)

    bmax = jnp.max(xw, axis=0, keepdims=True)
    rows = []
    for b in range(nseg):
        hit = (cu_ref[b] < start + tb) & (cu_ref[b + 1] > start)
        rows.append(jnp.where(hit, bmax, jnp.full_like(bmax, -1e30)))
    m_blk = jnp.concatenate(rows, axis=0)
    m_old = m_ref[...]
    m_new = jnp.maximum(m_old, m_blk)
    scale = jnp.exp(m_old - m_new)
    m_tok = jnp.dot(oh, m_new, preferred_element_type=jnp.float32)
    e = jnp.exp(xw - m_tok)
    p = e * xt
    contract = (((0,), (0,)), ((), ()))
    s_add = jax.lax.dot_general(oh, e, contract,
                                preferred_element_type=jnp.float32)
    n_add = jax.lax.dot_general(oh, p, contract,
                                preferred_element_type=jnp.float32)
    m_ref[...] = m_new
    s_new = s_ref[...] * scale + s_add
    n_new = n_ref[...] * scale + n_add
    s_ref[...] = s_new
    n_ref[...] = n_new

    @pl.when(step == nblocks - 1)
    def _fin():
        z = jnp.where(s_new > 0, n_new / s_new, 0.0)
        bb = jnp.sqrt(jnp.mean(bz * bz, axis=0, keepdims=True))
        mean = jnp.mean(z, axis=0, keepdims=True)
        var = jnp.sum((z - mean) ** 2, axis=0, keepdims=True) / (nseg - 1)
        std = jnp.sqrt(var)
        std = jnp.where(jnp.isnan(std), 1.0, std)
        out_ref[...] = bb * (z - mean) / std


@jax.jit
def _run(cu, x, bu2, bz2):
    t, d = x.shape
    ps = bu2.shape[1]
    nblocks = t // _TB
    body = functools.partial(_mipl_body, nblocks=nblocks, tb=_TB, nseg=_B)
    return pl.pallas_call(
        body,
        grid=(nblocks,),
        in_specs=[
            pl.BlockSpec(memory_space=pltpu.SMEM),
            pl.BlockSpec((_TB, d), lambda i: (i, 0)),
            pl.BlockSpec((d, ps), lambda i: (0, 0)),
            pl.BlockSpec((d, ps), lambda i: (0, 0)),
        ],
        out_specs=pl.BlockSpec((_B, ps), lambda i: (0, 0)),
        out_shape=jax.ShapeDtypeStruct((_B, ps), jnp.float32),
        scratch_shapes=[pltpu.VMEM((_B, ps), jnp.float32)] * 3,
        compiler_params=pltpu.CompilerParams(
            dimension_semantics=("arbitrary",)),
    )(cu, x, bu2, bz2)


def kernel(x, segment_ids, cu_seqlens, beta_u, beta_z):
    t, d = x.shape
    p, s = beta_u.shape[1], beta_u.shape[2]
    out = _run(cu_seqlens, x,
               beta_u.reshape(d, p * s), beta_z.reshape(d, p * s))
    return out.reshape(_B, p, s)


# speedup vs baseline: 1.6707x; 1.6707x over previous
import functools

import jax
import jax.numpy as jnp
from jax.experimental import pallas as pl
from jax.experimental.pallas import tpu as pltpu

_B = 16
_TB = 8192


def _mipl_body(cu_ref, x_ref, bu_ref, bz_ref,
               out_ref, m_ref, s_ref, n_ref, *, nblocks, tb, nseg):
    step = pl.program_id(0)

    @pl.when(step == 0)
    def _init():
        m_ref[...] = jnp.full_like(m_ref, -1e30)
        s_ref[...] = jnp.zeros_like(s_ref)
        n_ref[...] = jnp.zeros_like(n_ref)

    x = x_ref[...]
    bu = bu_ref[...]
    bz = bz_ref[...]
    eta = bz * jax.lax.rsqrt(jnp.mean(bz * bz, axis=0, keepdims=True))
    xw = jnp.dot(x, bu, preferred_element_type=jnp.float32)
    xt = jnp.dot(x, eta, preferred_element_type=jnp.float32)

    start = step * tb
    gidx = start + jax.lax.broadcasted_iota(jnp.int32, (tb, 1), 0)
    lo_row = jnp.concatenate(
        [jnp.full((1, 1), cu_ref[b], jnp.int32) for b in range(nseg)], axis=1)
    hi_row = jnp.concatenate(
        [jnp.full((1, 1), cu_ref[b + 1], jnp.int32) for b in range(nseg)],
        axis=1)
    oh = ((gidx >= lo_row) & (gidx < hi_row)).astype(jnp.float32)

    bmax = jnp.max(xw, axis=0, keepdims=True)
    rows = []
    for b in range(nseg):
        hit = (cu_ref[b] < start + tb) & (cu_ref[b + 1] > start)
        rows.append(jnp.where(hit, bmax, jnp.full_like(bmax, -1e30)))
    m_blk = jnp.concatenate(rows, axis=0)
    m_old = m_ref[...]
    m_new = jnp.maximum(m_old, m_blk)
    scale = jnp.exp(m_old - m_new)
    m_tok = jnp.dot(oh, m_new, preferred_element_type=jnp.float32)
    e = jnp.exp(xw - m_tok)
    p = e * xt
    contract = (((0,), (0,)), ((), ()))
    s_add = jax.lax.dot_general(oh, e, contract,
                                preferred_element_type=jnp.float32)
    n_add = jax.lax.dot_general(oh, p, contract,
                                preferred_element_type=jnp.float32)
    m_ref[...] = m_new
    s_new = s_ref[...] * scale + s_add
    n_new = n_ref[...] * scale + n_add
    s_ref[...] = s_new
    n_ref[...] = n_new

    @pl.when(step == nblocks - 1)
    def _fin():
        z = jnp.where(s_new > 0, n_new / s_new, 0.0)
        bb = jnp.sqrt(jnp.mean(bz * bz, axis=0, keepdims=True))
        mean = jnp.mean(z, axis=0, keepdims=True)
        var = jnp.sum((z - mean) ** 2, axis=0, keepdims=True) / (nseg - 1)
        std = jnp.sqrt(var)
        std = jnp.where(jnp.isnan(std), 1.0, std)
        out_ref[...] = bb * (z - mean) / std


@jax.jit
def _run(cu, x, bu2, bz2):
    t, d = x.shape
    ps = bu2.shape[1]
    nblocks = t // _TB
    body = functools.partial(_mipl_body, nblocks=nblocks, tb=_TB, nseg=_B)
    return pl.pallas_call(
        body,
        grid=(nblocks,),
        in_specs=[
            pl.BlockSpec(memory_space=pltpu.SMEM),
            pl.BlockSpec((_TB, d), lambda i: (i, 0)),
            pl.BlockSpec((d, ps), lambda i: (0, 0)),
            pl.BlockSpec((d, ps), lambda i: (0, 0)),
        ],
        out_specs=pl.BlockSpec((_B, ps), lambda i: (0, 0)),
        out_shape=jax.ShapeDtypeStruct((_B, ps), jnp.float32),
        scratch_shapes=[pltpu.VMEM((_B, ps), jnp.float32)] * 3,
        compiler_params=pltpu.CompilerParams(
            dimension_semantics=("arbitrary",)),
    )(cu, x, bu2, bz2)


def kernel(x, segment_ids, cu_seqlens, beta_u, beta_z):
    t, d = x.shape
    p, s = beta_u.shape[1], beta_u.shape[2]
    out = _run(cu_seqlens, x,
               beta_u.reshape(d, p * s), beta_z.reshape(d, p * s))
    return out.reshape(_B, p, s)


# speedup vs baseline: 1.7734x; 1.0614x over previous
import functools

import jax
import jax.numpy as jnp
from jax.experimental import pallas as pl
from jax.experimental.pallas import tpu as pltpu

_B = 16
_TB = 8192
_C = 128


def _mipl_body(cu_ref, x_ref, bu_ref, bz_ref,
               out_ref, m_ref, s_ref, n_ref, *, nblocks, tb, nseg):
    step = pl.program_id(0)

    @pl.when(step == 0)
    def _init():
        m_ref[...] = jnp.full_like(m_ref, -1e30)
        s_ref[...] = jnp.zeros_like(s_ref)
        n_ref[...] = jnp.zeros_like(n_ref)

    x = x_ref[...]
    bu = bu_ref[...]
    bz = bz_ref[...]
    msq = jnp.mean(bz * bz, axis=0, keepdims=True)
    eta = bz * jnp.where(msq > 0, jax.lax.rsqrt(msq), 0.0)
    xw = jnp.dot(x, bu, preferred_element_type=jnp.float32)
    xt = jnp.dot(x, eta, preferred_element_type=jnp.float32)

    start = step * tb
    gidx = start + jax.lax.broadcasted_iota(jnp.int32, (tb, nseg), 0)
    lo_row = jnp.concatenate(
        [jnp.full((1, 1), cu_ref[b], jnp.int32) for b in range(nseg)], axis=1)
    hi_row = jnp.concatenate(
        [jnp.full((1, 1), cu_ref[b + 1], jnp.int32) for b in range(nseg)],
        axis=1)
    oh = ((gidx >= lo_row) & (gidx < hi_row)).astype(jnp.float32)

    bmax = jnp.max(xw, axis=0, keepdims=True)
    rows = []
    for b in range(nseg):
        hit = (cu_ref[b] < start + tb) & (cu_ref[b + 1] > start)
        rows.append(jnp.where(hit, bmax, jnp.full_like(bmax, -1e30)))
    m_blk = jnp.concatenate(rows, axis=0)
    m_old = m_ref[...]
    m_new = jnp.maximum(m_old, m_blk)
    scale = jnp.exp(m_old - m_new)
    m_tok = jnp.dot(oh, m_new, preferred_element_type=jnp.float32)
    e = jnp.exp(xw - m_tok)
    ep = jnp.concatenate([e, e * xt], axis=1)
    sn_add = jax.lax.dot_general(oh, ep, (((0,), (0,)), ((), ())),
                                 preferred_element_type=jnp.float32)
    m_ref[...] = m_new
    s_new = s_ref[...] * scale + sn_add[:, :_C]
    n_new = n_ref[...] * scale + sn_add[:, _C:]
    s_ref[...] = s_new
    n_ref[...] = n_new

    @pl.when(step == nblocks - 1)
    def _fin():
        z = jnp.where(s_new > 0, n_new / s_new, 0.0)
        bb = jnp.sqrt(msq)
        mean = jnp.mean(z, axis=0, keepdims=True)
        var = jnp.sum((z - mean) ** 2, axis=0, keepdims=True) / (nseg - 1)
        std = jnp.sqrt(var)
        std = jnp.where(jnp.isnan(std), 1.0, std)
        out_ref[...] = bb * (z - mean) / std


@jax.jit
def _run(cu, x, bu2, bz2):
    t, d = x.shape
    nblocks = t // _TB
    body = functools.partial(_mipl_body, nblocks=nblocks, tb=_TB, nseg=_B)
    return pl.pallas_call(
        body,
        grid=(nblocks,),
        in_specs=[
            pl.BlockSpec(memory_space=pltpu.SMEM),
            pl.BlockSpec((_TB, d), lambda i: (i, 0)),
            pl.BlockSpec((d, _C), lambda i: (0, 0)),
            pl.BlockSpec((d, _C), lambda i: (0, 0)),
        ],
        out_specs=pl.BlockSpec((_B, _C), lambda i: (0, 0)),
        out_shape=jax.ShapeDtypeStruct((_B, _C), jnp.float32),
        scratch_shapes=[pltpu.VMEM((_B, _C), jnp.float32)] * 3,
        compiler_params=pltpu.CompilerParams(
            dimension_semantics=("arbitrary",)),
    )(cu, x, bu2, bz2)


def kernel(x, segment_ids, cu_seqlens, beta_u, beta_z):
    t, d = x.shape
    p, s = beta_u.shape[1], beta_u.shape[2]
    ps = p * s
    pad = ((0, 0), (0, _C - ps))
    out = _run(cu_seqlens, x,
               jnp.pad(beta_u.reshape(d, ps), pad),
               jnp.pad(beta_z.reshape(d, ps), pad))
    return out[:, :ps].reshape(_B, p, s)
